# baseline (speedup 1.0000x reference)
import numpy as np
import jax
import jax.numpy as jnp
from jax import lax
from jax.experimental import pallas as pl
from jax.experimental.pallas import tpu as pltpu
from jax.experimental.pallas import tpu_sc as plsc

NRP = 208
NW = 32
CH = 1568
NSLOT = 10016
NBB = 128
SFLAT = 16704
PQ = 144
NBLK = 13


def _finish_kernel(parts_ref, out_ref):
    out_ref[...] = jnp.sum(parts_ref[...], axis=0, keepdims=True)


def _make_sc_body(n_edges):
    n_full = n_edges // CH
    rem = n_edges - n_full * CH

    def body(ei_h, typ_h, u_h, v_h, lbl_h, rel_h, w_h, b_h, fcw_h, fill_h,
             fcb_h, out_h, tbl_h,
             slot_u, slot_v, gt_v, src_v, dst_v, typ_v, uv_v, vv_v, lbl_v,
             fcb_v, rel_v, w_v, b_v, f_v, vv6, sp, sq, ss, part_v, sem):
        c = lax.axis_index("c")
        s = lax.axis_index("s")
        wid = s * 2 + c
        base = wid * CH
        zf = jnp.zeros((16,), jnp.float32)
        iota16 = lax.broadcasted_iota(jnp.int32, (16,), 0)
        hs = [pltpu.async_copy(fill_h, slot_u, sem),
              pltpu.async_copy(fill_h, slot_v, sem),
              pltpu.async_copy(u_h, uv_v, sem),
              pltpu.async_copy(v_h, vv_v, sem),
              pltpu.async_copy(lbl_h, lbl_v, sem),
              pltpu.async_copy(fcb_h, fcb_v, sem)]

        nv16 = lax.select(wid < n_full, CH // 16, rem // 16)

        @pl.when(wid < n_full)
        def _copy_full():
            pltpu.sync_copy(ei_h.at[pl.ds(base, CH)], src_v)
            pltpu.sync_copy(ei_h.at[pl.ds(n_edges + base, CH)], dst_v)
            pltpu.sync_copy(typ_h.at[pl.ds(base, CH)], typ_v)

        if rem:
            @pl.when(wid == n_full)
            def _copy_tail():
                pltpu.sync_copy(ei_h.at[pl.ds(base, rem)],
                                src_v.at[pl.ds(0, rem)])
                pltpu.sync_copy(ei_h.at[pl.ds(n_edges + base, rem)],
                                dst_v.at[pl.ds(0, rem)])
                pltpu.sync_copy(typ_h.at[pl.ds(base, rem)],
                                typ_v.at[pl.ds(0, rem)])

        @pl.when(s < NBLK)
        def _table():
            pltpu.sync_copy(rel_h, rel_v.at[pl.ds(0, 32 * 200)])
            pltpu.sync_copy(w_h, w_v)
            pltpu.sync_copy(b_h, b_v)
            pltpu.sync_copy(fcw_h, f_v)
            r0 = s * 16

            def vstep(k, acc):
                fk = plsc.load_gather(f_v, [iota16 * 0 + k])
                outs = []
                for j in range(6):
                    wlo = w_v[pl.ds(j * 1024 + k * 32, 16)]
                    whi = w_v[pl.ds(j * 1024 + k * 32 + 16, 16)]
                    outs.append(acc[2 * j] + fk * wlo)
                    outs.append(acc[2 * j + 1] + fk * whi)
                return tuple(outs)
            vacc = lax.fori_loop(0, 32, vstep, tuple([zf] * 12))
            for j in range(6):
                vv6[pl.ds(j * 32, 16)] = vacc[2 * j]
                vv6[pl.ds(j * 32 + 16, 16)] = vacc[2 * j + 1]
            flo = f_v[pl.ds(0, 16)]
            fhi = f_v[pl.ds(16, 16)]
            betas = []
            for j in range(6):
                blo = b_v[pl.ds(j * 32, 16)]
                bhi = b_v[pl.ds(j * 32 + 16, 16)]
                betas.append(jnp.sum(flo * blo + fhi * bhi))

            ridx = (iota16 + r0) * 32

            def tstep(d, acc):
                relc = plsc.load_gather(rel_v, [ridx + d])
                fk = plsc.load_gather(f_v, [iota16 * 0 + d])
                outs = []
                for j in range(6):
                    vjd = plsc.load_gather(vv6, [iota16 * 0 + (j * 32 + d)])
                    outs.append(acc[j] + relc * vjd)
                outs.append(acc[6] + relc * fk)
                return tuple(outs)
            tacc = lax.fori_loop(0, 32, tstep, tuple([zf] * 7))
            t0 = tacc[0] + betas[0]
            t1 = tacc[1] + betas[1]
            t2 = tacc[2] + betas[2]
            t3 = tacc[3] + betas[3]
            t4 = tacc[4] + betas[4] - t1 - t2
            t5 = tacc[5] + betas[5] - t0 - t3
            rowbase = (iota16 + r0) * 8
            plsc.store_scatter(gt_v, [rowbase], t0)
            plsc.store_scatter(gt_v, [rowbase + 1], t1)
            plsc.store_scatter(gt_v, [rowbase + 2], t2)
            plsc.store_scatter(gt_v, [rowbase + 3], t3)
            plsc.store_scatter(gt_v, [rowbase + 4], t4)
            plsc.store_scatter(gt_v, [rowbase + 5], t5)
            plsc.store_scatter(gt_v, [rowbase + 6], tacc[6])
            plsc.store_scatter(gt_v, [rowbase + 7], zf)
            pltpu.sync_copy(gt_v.at[pl.ds(r0 * 8, 128)],
                            tbl_h.at[pl.ds(r0 * 8, 128)])
        plsc.subcore_barrier()
        pltpu.sync_copy(tbl_h, gt_v)

        for h in hs:
            h.wait()

        for k in range(PQ // 16):
            sp[pl.ds(k * 16, 16)] = zf
            sq[pl.ds(k * 16, 16)] = zf

        for k in range(NBB // 16):
            ub = uv_v[pl.ds(k * 16, 16)]
            vb = vv_v[pl.ds(k * 16, 16)]
            plsc.store_scatter(slot_u, [ub], iota16 + (k * 16))
            plsc.store_scatter(slot_v, [vb], iota16 + (k * 16))

        for k in range(NBB // 16):
            ub = uv_v[pl.ds(k * 16, 16)]
            vb = vv_v[pl.ds(k * 16, 16)]
            su_b = plsc.load_gather(slot_u, [ub])
            sv_b = plsc.load_gather(slot_v, [vb])
            plsc.store_scatter(ss, [su_b * 129 + sv_b], zf)

        def scatter16(off):
            s16 = src_v[pl.ds(off, 16)]
            d16 = dst_v[pl.ds(off, 16)]
            t16 = typ_v[pl.ds(off, 16)]
            su_s = plsc.load_gather(slot_u, [s16])
            su_d = plsc.load_gather(slot_u, [d16])
            sv_s = plsc.load_gather(slot_v, [s16])
            sv_d = plsc.load_gather(slot_v, [d16])
            m_us = su_s < NBB
            m_ud = su_d < NBB
            m_vs = sv_s < NBB
            m_vd = sv_d < NBB
            t8 = t16 * 8
            g0 = plsc.load_gather(gt_v, [t8])
            g1 = plsc.load_gather(gt_v, [t8 + 1])
            g2 = plsc.load_gather(gt_v, [t8 + 2])
            g3 = plsc.load_gather(gt_v, [t8 + 3])
            g4 = plsc.load_gather(gt_v, [t8 + 4])
            g5 = plsc.load_gather(gt_v, [t8 + 5])
            plsc.addupdate_scatter(sp, [su_d], g0, mask=m_ud)
            plsc.addupdate_scatter(sp, [su_s], g1, mask=m_us)
            plsc.addupdate_scatter(sq, [sv_d], g2, mask=m_vd)
            plsc.addupdate_scatter(sq, [sv_s], g3, mask=m_vs)
            plsc.addupdate_scatter(ss, [su_s * 129 + sv_d], g4,
                                   mask=m_us & m_vd)
            plsc.addupdate_scatter(ss, [su_d * 129 + sv_s], g5,
                                   mask=m_ud & m_vs)

        def edge_step4(i, carry):
            scatter16(i * 64)
            scatter16(i * 64 + 16)
            scatter16(i * 64 + 32)
            scatter16(i * 64 + 48)
            return carry
        lax.fori_loop(0, nv16 // 4, edge_step4, 0)

        def edge_step1(i, carry):
            scatter16(i * 16)
            return carry
        lax.fori_loop((nv16 // 4) * 4, nv16, edge_step1, 0)

        rflag = (wid == 0).astype(jnp.float32)
        for k in range(NBB // 16):
            ub = uv_v[pl.ds(k * 16, 16)]
            vb = vv_v[pl.ds(k * 16, 16)]
            lb = lbl_v[pl.ds(k * 16, 16)]
            su_b = plsc.load_gather(slot_u, [ub])
            sv_b = plsc.load_gather(slot_v, [vb])
            pv = plsc.load_gather(sp, [su_b])
            qv = plsc.load_gather(sq, [sv_b])
            sv = plsc.load_gather(ss, [su_b * 129 + sv_b])
            rterm = plsc.load_gather(gt_v, [lb * 8 + 6])
            fcb_sp = plsc.load_gather(fcb_v, [iota16 * 0])
            part_v[pl.ds(k * 16, 16)] = (pv + qv + sv
                                         + (rterm + fcb_sp) * rflag)
        pltpu.sync_copy(part_v, out_h.at[wid])
    return body


def kernel(edge_index, edge_type, target_u, target_v, rel_labels,
           rel_emb_weight, W_reld2, b_reld2, fc_W, fc_b):
    e0 = edge_type.shape[0]
    eflat = edge_index.astype(jnp.int32).reshape(-1)
    typ = edge_type.astype(jnp.int32)
    fill = jnp.asarray(np.full((NSLOT,), NBB, np.int32))
    u1 = target_u.astype(jnp.int32)
    v1 = target_v.astype(jnp.int32)
    l1 = rel_labels.astype(jnp.int32)
    rel_f = rel_emb_weight.reshape(-1)
    w_f = W_reld2.reshape(-1)
    b_f = b_reld2.reshape(-1)
    fcw_f = fc_W.reshape(-1)

    mesh = plsc.VectorSubcoreMesh(core_axis_name="c", subcore_axis_name="s")
    parts, _ = pl.kernel(
        _make_sc_body(e0),
        out_type=(jax.ShapeDtypeStruct((NW, NBB), jnp.float32),
                  jax.ShapeDtypeStruct((8 * NRP,), jnp.float32)),
        mesh=mesh,
        compiler_params=pltpu.CompilerParams(needs_layout_passes=False),
        scratch_types=[
            pltpu.VMEM((NSLOT,), jnp.int32),
            pltpu.VMEM((NSLOT,), jnp.int32),
            pltpu.VMEM((8 * NRP,), jnp.float32),
            pltpu.VMEM((CH,), jnp.int32),
            pltpu.VMEM((CH,), jnp.int32),
            pltpu.VMEM((CH,), jnp.int32),
            pltpu.VMEM((NBB,), jnp.int32),
            pltpu.VMEM((NBB,), jnp.int32),
            pltpu.VMEM((NBB,), jnp.int32),
            pltpu.VMEM((1,), jnp.float32),
            pltpu.VMEM((32 * NRP,), jnp.float32),
            pltpu.VMEM((6 * 32 * 32,), jnp.float32),
            pltpu.VMEM((192,), jnp.float32),
            pltpu.VMEM((32,), jnp.float32),
            pltpu.VMEM((192,), jnp.float32),
            pltpu.VMEM((PQ,), jnp.float32),
            pltpu.VMEM((PQ,), jnp.float32),
            pltpu.VMEM((SFLAT,), jnp.float32),
            pltpu.VMEM((NBB,), jnp.float32),
            pltpu.SemaphoreType.DMA,
        ],
    )(eflat, typ, u1, v1, l1, rel_f, w_f, b_f, fcw_f, fill, fc_b)

    out = pl.pallas_call(
        _finish_kernel,
        in_specs=[
            pl.BlockSpec((NW, NBB), lambda: (0, 0)),
        ],
        out_specs=pl.BlockSpec((1, NBB), lambda: (0, 0)),
        out_shape=jax.ShapeDtypeStruct((1, NBB), jnp.float32),
    )(parts)
    return out.reshape(NBB, 1)


# speedup vs baseline: 1.0825x; 1.0825x over previous
import numpy as np
import jax
import jax.numpy as jnp
from jax import lax
from jax.experimental import pallas as pl
from jax.experimental.pallas import tpu as pltpu
from jax.experimental.pallas import tpu_sc as plsc

NRP = 208
NW = 32
CH = 1568
NSLOT = 10016
NBB = 128
SFLAT = 16704
PQ = 144


def _prep_kernel(relaug_ref, waug_ref, fcw_ref, fixt_ref, gt_ref, vt_ref):
    f = fcw_ref[0:1, :]
    for j in range(6):
        wj = waug_ref[32 * j:32 * j + 32, :]
        vt_ref[j:j + 1, :] = jax.lax.dot_general(
            f, wj, (((1,), (0,)), ((), ())))
    f40 = jnp.concatenate([f, jnp.zeros((1, 8), jnp.float32)], axis=1)
    vt_ref[6:7, :] = f40
    vt_ref[7:8, :] = jnp.zeros((1, 40), jnp.float32)
    gt8 = jax.lax.dot_general(
        vt_ref[...], relaug_ref[...], (((1,), (1,)), ((), ())))
    gt_ref[...] = jax.lax.dot_general(
        fixt_ref[...], gt8, (((1,), (0,)), ((), ())))


def _finish_kernel(parts_ref, out_ref):
    out_ref[...] = jnp.sum(parts_ref[...], axis=0, keepdims=True)


def _make_sc_body(n_edges):
    n_full = n_edges // CH
    rem = n_edges - n_full * CH

    def body(ei_h, typ_h, u_h, v_h, lbl_h, gt_h, fill_h, fcb_h, out_h,
             slot_u, slot_v, gt_v, src_v, dst_v, typ_v, uv_v, vv_v, lbl_v,
             fcb_v, sp, sq, ss, part_v, sem):
        c = lax.axis_index("c")
        s = lax.axis_index("s")
        wid = s * 2 + c
        base = wid * CH
        hs = [pltpu.async_copy(fill_h, slot_u, sem),
              pltpu.async_copy(fill_h, slot_v, sem),
              pltpu.async_copy(u_h, uv_v, sem),
              pltpu.async_copy(v_h, vv_v, sem),
              pltpu.async_copy(lbl_h, lbl_v, sem),
              pltpu.async_copy(gt_h, gt_v, sem),
              pltpu.async_copy(fcb_h, fcb_v, sem)]

        nv16 = lax.select(wid < n_full, CH // 16, rem // 16)

        @pl.when(wid < n_full)
        def _copy_full():
            pltpu.sync_copy(ei_h.at[pl.ds(base, CH)], src_v)
            pltpu.sync_copy(ei_h.at[pl.ds(n_edges + base, CH)], dst_v)
            pltpu.sync_copy(typ_h.at[pl.ds(base, CH)], typ_v)

        if rem:
            @pl.when(wid == n_full)
            def _copy_tail():
                pltpu.sync_copy(ei_h.at[pl.ds(base, rem)],
                                src_v.at[pl.ds(0, rem)])
                pltpu.sync_copy(ei_h.at[pl.ds(n_edges + base, rem)],
                                dst_v.at[pl.ds(0, rem)])
                pltpu.sync_copy(typ_h.at[pl.ds(base, rem)],
                                typ_v.at[pl.ds(0, rem)])

        for h in hs:
            h.wait()

        zf = jnp.zeros((16,), jnp.float32)
        iota16 = lax.broadcasted_iota(jnp.int32, (16,), 0)

        for k in range(PQ // 16):
            sp[pl.ds(k * 16, 16)] = zf
            sq[pl.ds(k * 16, 16)] = zf

        for k in range(NBB // 16):
            ub = uv_v[pl.ds(k * 16, 16)]
            vb = vv_v[pl.ds(k * 16, 16)]
            plsc.store_scatter(slot_u, [ub], iota16 + (k * 16))
            plsc.store_scatter(slot_v, [vb], iota16 + (k * 16))

        for k in range(NBB // 16):
            ub = uv_v[pl.ds(k * 16, 16)]
            vb = vv_v[pl.ds(k * 16, 16)]
            su_b = plsc.load_gather(slot_u, [ub])
            sv_b = plsc.load_gather(slot_v, [vb])
            plsc.store_scatter(ss, [su_b * 129 + sv_b], zf)

        def scatter16(off):
            s16 = src_v[pl.ds(off, 16)]
            d16 = dst_v[pl.ds(off, 16)]
            t16 = typ_v[pl.ds(off, 16)]
            su_s = plsc.load_gather(slot_u, [s16])
            su_d = plsc.load_gather(slot_u, [d16])
            sv_s = plsc.load_gather(slot_v, [s16])
            sv_d = plsc.load_gather(slot_v, [d16])
            m_us = su_s < NBB
            m_ud = su_d < NBB
            m_vs = sv_s < NBB
            m_vd = sv_d < NBB
            g0 = plsc.load_gather(gt_v, [t16])
            g1 = plsc.load_gather(gt_v, [t16 + NRP])
            g2 = plsc.load_gather(gt_v, [t16 + 2 * NRP])
            g3 = plsc.load_gather(gt_v, [t16 + 3 * NRP])
            g4 = plsc.load_gather(gt_v, [t16 + 4 * NRP])
            g5 = plsc.load_gather(gt_v, [t16 + 5 * NRP])
            plsc.addupdate_scatter(sp, [su_d], g0, mask=m_ud)
            plsc.addupdate_scatter(sp, [su_s], g1, mask=m_us)
            plsc.addupdate_scatter(sq, [sv_d], g2, mask=m_vd)
            plsc.addupdate_scatter(sq, [sv_s], g3, mask=m_vs)
            plsc.addupdate_scatter(ss, [su_s * 129 + sv_d], g4,
                                   mask=m_us & m_vd)
            plsc.addupdate_scatter(ss, [su_d * 129 + sv_s], g5,
                                   mask=m_ud & m_vs)

        def edge_step4(i, carry):
            scatter16(i * 64)
            scatter16(i * 64 + 16)
            scatter16(i * 64 + 32)
            scatter16(i * 64 + 48)
            return carry
        lax.fori_loop(0, nv16 // 4, edge_step4, 0)

        def edge_step1(i, carry):
            scatter16(i * 16)
            return carry
        lax.fori_loop((nv16 // 4) * 4, nv16, edge_step1, 0)

        rflag = (wid == 0).astype(jnp.float32)
        for k in range(NBB // 16):
            ub = uv_v[pl.ds(k * 16, 16)]
            vb = vv_v[pl.ds(k * 16, 16)]
            lb = lbl_v[pl.ds(k * 16, 16)]
            su_b = plsc.load_gather(slot_u, [ub])
            sv_b = plsc.load_gather(slot_v, [vb])
            pv = plsc.load_gather(sp, [su_b])
            qv = plsc.load_gather(sq, [sv_b])
            sv = plsc.load_gather(ss, [su_b * 129 + sv_b])
            rterm = plsc.load_gather(gt_v, [lb + 6 * NRP])
            fcb_sp = plsc.load_gather(fcb_v, [iota16 * 0])
            part_v[pl.ds(k * 16, 16)] = (pv + qv + sv
                                         + (rterm + fcb_sp) * rflag)
        pltpu.sync_copy(part_v, out_h.at[wid])
    return body


def kernel(edge_index, edge_type, target_u, target_v, rel_labels,
           rel_emb_weight, W_reld2, b_reld2, fc_W, fc_b):
    e0 = edge_type.shape[0]
    eflat = edge_index.astype(jnp.int32).reshape(-1)
    typ = edge_type.astype(jnp.int32)
    fill = jnp.asarray(np.full((NSLOT,), NBB, np.int32))
    u1 = target_u.astype(jnp.int32)
    v1 = target_v.astype(jnp.int32)
    l1 = rel_labels.astype(jnp.int32)
    nr = rel_emb_weight.shape[0]
    relaug = jnp.pad(
        jnp.concatenate([rel_emb_weight,
                         jnp.ones((nr, 1), jnp.float32)], axis=1),
        ((0, NRP - nr), (0, 7)))
    waug = jnp.concatenate(
        [W_reld2, b_reld2[:, :, None],
         jnp.zeros((6, 32, 7), jnp.float32)], axis=2).reshape(192, 40)
    fcwp = jnp.pad(fc_W, ((0, 7), (0, 0)))
    fix = np.eye(8, dtype=np.float32)
    fix[1, 4] = fix[2, 4] = -1.0
    fix[0, 5] = fix[3, 5] = -1.0
    fixt = jnp.asarray(fix.T)

    gt = pl.pallas_call(
        _prep_kernel,
        in_specs=[
            pl.BlockSpec((NRP, 40), lambda: (0, 0)),
            pl.BlockSpec((192, 40), lambda: (0, 0)),
            pl.BlockSpec((8, 32), lambda: (0, 0)),
            pl.BlockSpec((8, 8), lambda: (0, 0)),
        ],
        out_specs=pl.BlockSpec((8, NRP), lambda: (0, 0)),
        out_shape=jax.ShapeDtypeStruct((8, NRP), jnp.float32),
        scratch_shapes=[pltpu.VMEM((8, 40), jnp.float32)],
    )(relaug, waug, fcwp, fixt)
    gtflat = gt.reshape(8 * NRP)

    mesh = plsc.VectorSubcoreMesh(core_axis_name="c", subcore_axis_name="s")
    parts = pl.kernel(
        _make_sc_body(e0),
        out_type=jax.ShapeDtypeStruct((NW, NBB), jnp.float32),
        mesh=mesh,
        compiler_params=pltpu.CompilerParams(needs_layout_passes=False),
        scratch_types=[
            pltpu.VMEM((NSLOT,), jnp.int32),
            pltpu.VMEM((NSLOT,), jnp.int32),
            pltpu.VMEM((8 * NRP,), jnp.float32),
            pltpu.VMEM((CH,), jnp.int32),
            pltpu.VMEM((CH,), jnp.int32),
            pltpu.VMEM((CH,), jnp.int32),
            pltpu.VMEM((NBB,), jnp.int32),
            pltpu.VMEM((NBB,), jnp.int32),
            pltpu.VMEM((NBB,), jnp.int32),
            pltpu.VMEM((1,), jnp.float32),
            pltpu.VMEM((PQ,), jnp.float32),
            pltpu.VMEM((PQ,), jnp.float32),
            pltpu.VMEM((SFLAT,), jnp.float32),
            pltpu.VMEM((NBB,), jnp.float32),
            pltpu.SemaphoreType.DMA,
        ],
    )(eflat, typ, u1, v1, l1, gtflat, fill, fc_b)

    out = pl.pallas_call(
        _finish_kernel,
        in_specs=[
            pl.BlockSpec((NW, NBB), lambda: (0, 0)),
        ],
        out_specs=pl.BlockSpec((1, NBB), lambda: (0, 0)),
        out_shape=jax.ShapeDtypeStruct((1, NBB), jnp.float32),
    )(parts)
    return out.reshape(NBB, 1)
